# baseline (speedup 1.0000x reference)
import functools
import math

import jax
import jax.numpy as jnp
from jax import lax
from jax.experimental import pallas as pl
from jax.experimental.pallas import tpu as pltpu
from jax.experimental.pallas import tpu_sc as plsc

NS, NG, DC, DG, H, GDF, K = 10000, 2500, 128, 16, 256, 16, 16
E = NS * K
BNS = 1.0 / math.sqrt(1.0 + 1e-5)

_SC_CORES, _SC_SUBCORES = 2, 16
_NW = _SC_CORES * _SC_SUBCORES


def _bn(x, g, b):
    return x * (BNS * g) + b


def _silu(x):
    return x * jax.nn.sigmoid(x)


def _chem_body(chem_ref, cw1_ref, cb1_ref, cg1_ref, cbe1_ref, cw2_ref, cb2_ref,
               cg2_ref, cbe2_ref, sw1c_ref, sb1_ref, hchem_ref, proj_ref):
    x = chem_ref[...]
    h = _bn(jnp.dot(x, cw1_ref[...], preferred_element_type=jnp.float32)
            + cb1_ref[...], cg1_ref[...], cbe1_ref[...])
    h = _silu(h)
    h = _bn(jnp.dot(h, cw2_ref[...], preferred_element_type=jnp.float32)
            + cb2_ref[...], cg2_ref[...], cbe2_ref[...])
    hchem_ref[...] = h
    proj_ref[...] = jnp.dot(x, sw1c_ref[...], preferred_element_type=jnp.float32)


def _chem_prep(chem_feats, cw1, cb1, cg1, cbe1, cw2, cb2, cg2, cbe2, sw1c, sb1):
    return pl.pallas_call(
        _chem_body,
        out_shape=(jax.ShapeDtypeStruct((NG, H), jnp.float32),
                   jax.ShapeDtypeStruct((NG, H), jnp.float32)),
    )(chem_feats, cw1, cb1, cg1, cbe1, cw2, cb2, cg2, cbe2, sw1c, sb1)


NGP = 2560


def _geom_body(geom_ref, verts_ref, npT_ref, pad_ref, npTt_ref, padt_ref,
               gw1_ref, gb1_ref, gg1_ref, gbe1_ref, gw2_ref, gb2_ref, gg2_ref,
               gbe2_ref, hg_ref, d2_ref, d2t_ref, ang_ref):
    x = geom_ref[...]
    h = _bn(jnp.dot(x, gw1_ref[...], preferred_element_type=jnp.float32)
            + gb1_ref[...], gg1_ref[...], gbe1_ref[...])
    h = _silu(h)
    h = _bn(jnp.dot(h, gw2_ref[...], preferred_element_type=jnp.float32)
            + gb2_ref[...], gg2_ref[...], gbe2_ref[...])
    hg_ref[...] = h
    v = verts_ref[...]
    npT = npT_ref[...]
    vsq = jnp.sum(v * v, axis=1, keepdims=True)
    nsq = jnp.sum(npT * npT, axis=0, keepdims=True) + pad_ref[...]
    d2_ref[...] = vsq + nsq - 2.0 * jnp.dot(v, npT,
                                            preferred_element_type=jnp.float32)
    npTt = npTt_ref[...]
    nsqt = jnp.sum(npTt * npTt, axis=0, keepdims=True) + padt_ref[...]
    d2t_ref[...] = vsq + nsqt - 2.0 * jnp.dot(v, npTt,
                                              preferred_element_type=jnp.float32)
    n = x[:, DG - 3:]
    nv = jnp.sum(n * v, axis=1, keepdims=True)
    ang_ref[...] = jnp.dot(n, npT,
                           preferred_element_type=jnp.float32) - nv


def _geom_prep(geom_feats, verts, npT, pad, npTt, padt, gw1, gb1, gg1, gbe1,
               gw2, gb2, gg2, gbe2):
    B = 400
    grid = NS // B
    return pl.pallas_call(
        _geom_body,
        grid=(grid,),
        in_specs=[
            pl.BlockSpec((B, DG), lambda i: (i, 0)),
            pl.BlockSpec((B, 3), lambda i: (i, 0)),
            pl.BlockSpec((3, NGP), lambda i: (0, 0)),
            pl.BlockSpec((1, NGP), lambda i: (0, 0)),
            pl.BlockSpec((3, NGP), lambda i: (0, 0)),
            pl.BlockSpec((1, NGP), lambda i: (0, 0)),
            pl.BlockSpec((DG, H), lambda i: (0, 0)),
            pl.BlockSpec((1, H), lambda i: (0, 0)),
            pl.BlockSpec((1, H), lambda i: (0, 0)),
            pl.BlockSpec((1, H), lambda i: (0, 0)),
            pl.BlockSpec((H, H), lambda i: (0, 0)),
            pl.BlockSpec((1, H), lambda i: (0, 0)),
            pl.BlockSpec((1, H), lambda i: (0, 0)),
            pl.BlockSpec((1, H), lambda i: (0, 0)),
        ],
        out_specs=[
            pl.BlockSpec((B, H), lambda i: (i, 0)),
            pl.BlockSpec((B, NGP), lambda i: (i, 0)),
            pl.BlockSpec((B, NGP), lambda i: (i, 0)),
            pl.BlockSpec((B, NGP), lambda i: (i, 0)),
        ],
        out_shape=(jax.ShapeDtypeStruct((NS, H), jnp.float32),
                   jax.ShapeDtypeStruct((NS, NGP), jnp.float32),
                   jax.ShapeDtypeStruct((NS, NGP), jnp.float32),
                   jax.ShapeDtypeStruct((NS, NGP), jnp.float32)),
    )(geom_feats, verts, npT, pad, npTt, padt, gw1, gb1, gg1, gbe1, gw2, gb2,
      gg2, gbe2)


_TG = 4
_NGRP = NS // _TG
_CPV = NGP // 16


def _lexmin(v1, c1, v2, c2):
    take = (v2 < v1) | ((v2 == v1) & (c2 < c1))
    return jnp.where(take, v2, v1), jnp.where(take, c2, c1)


_CL = _CPV // 2


def _sc_topk_body(d2_ref, d2t_ref, ang_ref, idx_ref, d2e_ref, ange_ref,
                  buf0, buft0, bufa0, buf1, buft1, bufa1,
                  stage, staged, stagea,
                  sem0, semt0, sema0, sem1, semt1, sema1):
    wid = lax.axis_index("s") * _SC_CORES + lax.axis_index("c")
    lane = lax.iota(jnp.int32, 16)
    inf16 = jnp.full((16,), jnp.inf, jnp.float32)
    big16 = jnp.full((16,), NGP, jnp.int32)
    sets = ((buf0, buft0, bufa0, sem0, semt0, sema0),
            (buf1, buft1, bufa1, sem1, semt1, sema1))

    def start(gi, p):
        buf, buft, bufa, sem, semt, sema = sets[p]
        pltpu.async_copy(d2_ref.at[pl.ds(gi * (_TG * NGP), _TG * NGP)], buf,
                         sem)
        pltpu.async_copy(d2t_ref.at[pl.ds(gi * (_TG * NGP), _TG * NGP)], buft,
                         semt)
        pltpu.async_copy(ang_ref.at[pl.ds(gi * (_TG * NGP), _TG * NGP)], bufa,
                         sema)

    def do_group(gi, p):
        buf, buft, bufa, sem, semt, sema = sets[p]
        pltpu.make_async_copy(
            d2_ref.at[pl.ds(gi * (_TG * NGP), _TG * NGP)], buf, sem).wait()
        pltpu.make_async_copy(
            d2t_ref.at[pl.ds(gi * (_TG * NGP), _TG * NGP)], buft, semt).wait()
        pltpu.make_async_copy(
            ang_ref.at[pl.ds(gi * (_TG * NGP), _TG * NGP)], bufa, sema).wait()
        def do_row(rr, rcarry):
            roff = rr * NGP

            def p1(t, carry):
                a0, s0, a1, s1 = carry
                j0 = 4 * t
                for u in (0, 2):
                    v0 = buf[pl.ds(roff + (j0 + u) * 16, 16)]
                    c0 = v0 < a0
                    a0 = jnp.where(c0, v0, a0)
                    s0 = jnp.where(c0, j0 + u, s0)
                    v1 = buf[pl.ds(roff + (j0 + u + 1) * 16, 16)]
                    c1 = v1 < a1
                    a1 = jnp.where(c1, v1, a1)
                    s1 = jnp.where(c1, j0 + u + 1, s1)
                return a0, s0, a1, s1

            a0, s0, a1, s1 = lax.fori_loop(
                0, _CPV // 4, p1, (inf16, big16, inf16, big16))

            def sel(k, carry):
                a0, s0, a1, s1, outv, outd, outa = carry
                val, col = _lexmin(a0, s0 * 16 + lane, a1, s1 * 16 + lane)
                for s in (1, 2, 4, 8):
                    pp = lane ^ s
                    v2 = val.at[pp].get(mode="promise_in_bounds")
                    c2 = col.at[pp].get(mode="promise_in_bounds")
                    val, col = _lexmin(val, col, v2, c2)
                outv = jnp.where(lane == k, col, outv)
                outd = jnp.where(lane == k, val, outd)
                av = bufa[pl.ds(roff + (col[0] >> 4) * 16, 16)]
                al = av.at[col & 15].get(mode="promise_in_bounds")
                outa = jnp.where(lane == k, al, outa)
                ls = col[0] & 15
                jcol = col >> 4
                ps = col[0] >> 4 & 1
                coff = roff + (ps * 16 + ls) * _CL

                def rf(t, carry2):
                    a, aj = carry2
                    w = buft[pl.ds(coff + t * 16, 16)]
                    jv = (t * 16 + lane) * 2 + ps
                    elig = (w > val) | ((w == val) & (jv > jcol))
                    w = jnp.where(elig, w, jnp.inf)
                    return _lexmin(a, aj, w, jv)

                a, aj = lax.fori_loop(0, _CL // 16, rf, (inf16, big16))
                for s in (1, 2, 4, 8):
                    pp = lane ^ s
                    v2 = a.at[pp].get(mode="promise_in_bounds")
                    c2 = aj.at[pp].get(mode="promise_in_bounds")
                    a, aj = _lexmin(a, aj, v2, c2)
                upd = jnp.where(lane == ls, ps + 1, 0)
                a0 = jnp.where(upd == 1, a, a0)
                s0 = jnp.where(upd == 1, aj, s0)
                a1 = jnp.where(upd == 2, a, a1)
                s1 = jnp.where(upd == 2, aj, s1)
                return a0, s0, a1, s1, outv, outd, outa

            carry = (a0, s0, a1, s1, jnp.zeros((16,), jnp.int32),
                     jnp.zeros((16,), jnp.float32), jnp.zeros((16,), jnp.float32))
            res = lax.fori_loop(0, K, sel, carry)
            stage[pl.ds(rr * K, K)] = res[4]
            staged[pl.ds(rr * K, K)] = res[5]
            stagea[pl.ds(rr * K, K)] = res[6]
            return rcarry

        lax.fori_loop(0, _TG, do_row, 0)
        pltpu.sync_copy(stage, idx_ref.at[pl.ds(gi * (_TG * K), _TG * K)])
        pltpu.sync_copy(staged, d2e_ref.at[pl.ds(gi * (_TG * K), _TG * K)])
        pltpu.sync_copy(stagea, ange_ref.at[pl.ds(gi * (_TG * K), _TG * K)])

    nloop = (_NGRP + _NW - 1) // _NW

    @pl.when(wid < _NGRP)
    def _():
        start(wid, 0)

    def loop_body(t, carry):
        for p in (0, 1):
            j = 2 * t + p
            gi = wid + j * _NW
            gin = wid + (j + 1) * _NW

            @pl.when(gin < _NGRP)
            def _():
                start(gin, 1 - p)

            @pl.when(gi < _NGRP)
            def _():
                do_group(gi, p)

        return carry

    lax.fori_loop(0, (nloop + 1) // 2, loop_body, 0)


def _sc_topk(d2_flat, d2t_flat, ang_flat):
    return pl.kernel(
        _sc_topk_body,
        out_type=(jax.ShapeDtypeStruct((E,), jnp.int32),
                  jax.ShapeDtypeStruct((E,), jnp.float32),
                  jax.ShapeDtypeStruct((E,), jnp.float32)),
        mesh=plsc.VectorSubcoreMesh(core_axis_name="c", subcore_axis_name="s",
                                    num_cores=_SC_CORES,
                                    num_subcores=_SC_SUBCORES),
        scratch_types=[pltpu.VMEM((_TG * NGP,), jnp.float32),
                       pltpu.VMEM((_TG * NGP,), jnp.float32),
                       pltpu.VMEM((_TG * NGP,), jnp.float32),
                       pltpu.VMEM((_TG * NGP,), jnp.float32),
                       pltpu.VMEM((_TG * NGP,), jnp.float32),
                       pltpu.VMEM((_TG * NGP,), jnp.float32),
                       pltpu.VMEM((_TG * K,), jnp.int32),
                       pltpu.VMEM((_TG * K,), jnp.float32),
                       pltpu.VMEM((_TG * K,), jnp.float32),
                       pltpu.SemaphoreType.DMA,
                       pltpu.SemaphoreType.DMA,
                       pltpu.SemaphoreType.DMA,
                       pltpu.SemaphoreType.DMA,
                       pltpu.SemaphoreType.DMA,
                       pltpu.SemaphoreType.DMA],
        compiler_params=pltpu.CompilerParams(use_tc_tiling_on_sc=False),
    )(d2_flat, d2t_flat, ang_flat)


_GC = 128


def _sc_gather_body(tblc_ref, idx_ref, chem_ref, idx_v, chem_v, sem1):
    wid = lax.axis_index("s") * _SC_CORES + lax.axis_index("c")
    nchunk = idx_ref.shape[0] // _GC

    def do_chunk(ci):
        base = ci * _GC
        pltpu.sync_copy(idx_ref.at[pl.ds(base, _GC)], idx_v)
        pltpu.async_copy(tblc_ref.at[idx_v], chem_v, sem1).wait()
        pltpu.sync_copy(chem_v, chem_ref.at[pl.ds(base, _GC)])

    def loop_body(j, carry):
        do_chunk(wid + j * _NW)
        return carry

    lax.fori_loop(0, nchunk // _NW, loop_body, 0)
    rem = nchunk - (nchunk // _NW) * _NW

    @pl.when(wid < rem)
    def _():
        do_chunk((nchunk // _NW) * _NW + wid)


def _sc_gather(tblc, idx_flat):
    return pl.kernel(
        _sc_gather_body,
        out_type=jax.ShapeDtypeStruct((idx_flat.shape[0], 64), jnp.int32),
        mesh=plsc.VectorSubcoreMesh(core_axis_name="c", subcore_axis_name="s",
                                    num_cores=_SC_CORES,
                                    num_subcores=_SC_SUBCORES),
        scratch_types=[pltpu.VMEM((_GC,), jnp.int32),
                       pltpu.VMEM((_GC, 64), jnp.int32),
                       pltpu.SemaphoreType.DMA],
        compiler_params=pltpu.CompilerParams(use_tc_tiling_on_sc=False),
    )(tblc, idx_flat)


def _edge_body(chem_g_ref, d2k_ref, angk_ref, hg1_ref, mu_ref,
               sw1_ref, sb1_ref, sg1_ref, sbe1_ref, sw2_ref, sb2_ref,
               sg2_ref, sbe2_ref, fw1_ref, fb1_ref, fg1_ref, fbe1_ref,
               fw2_ref, fb2_ref, fg2_ref, fbe2_ref, out_ref):
    BV = d2k_ref.shape[0]
    EB = BV * K

    dist = jnp.sqrt(jnp.maximum(d2k_ref[...], 1e-30))
    ang = angk_ref[...] / dist

    mu_d = mu_ref[0:1, :].reshape(1, 1, GDF)
    mu_a = mu_ref[1:2, :].reshape(1, 1, GDF)
    enc_d = jnp.exp(-(((dist[:, :, None] - mu_d) / 0.5) ** 2)).reshape(EB, GDF)
    enc_a = jnp.exp(-(((ang[:, :, None] - mu_a) / 0.125) ** 2)).reshape(EB, GDF)

    enc = jnp.concatenate([enc_d, enc_a], axis=1)
    x1 = (jnp.dot(chem_g_ref[...], sw1_ref[:DC, :].astype(jnp.bfloat16),
                  preferred_element_type=jnp.float32)
          + jnp.dot(enc, sw1_ref[DC:, :], preferred_element_type=jnp.float32))
    h = _bn(x1 + sb1_ref[...], sg1_ref[...], sbe1_ref[...])
    h = _silu(h)
    h = _bn(jnp.dot(h.astype(jnp.bfloat16), sw2_ref[...].astype(jnp.bfloat16),
                    preferred_element_type=jnp.float32)
            + sb2_ref[...], sg2_ref[...], sbe2_ref[...])
    filt, core = h[:, :H], h[:, H:]
    he = jax.nn.sigmoid(filt) * jax.nn.softplus(core)
    h_cg = jnp.sum(he.reshape(BV, K, H), axis=1)

    y = jnp.concatenate([h_cg, hg1_ref[...]], axis=1)
    y = _bn(jnp.dot(y, fw1_ref[...], preferred_element_type=jnp.float32)
            + fb1_ref[...], fg1_ref[...], fbe1_ref[...])
    y = _silu(y)
    y = _bn(jnp.dot(y, fw2_ref[...], preferred_element_type=jnp.float32)
            + fb2_ref[...], fg2_ref[...], fbe2_ref[...])
    out_ref[...] = y


def _edge_final(chem_g, d2k, angk, hg1, mu, sw1, sb1, sg1, sbe1, sw2,
                sb2, sg2, sbe2, fw1, fb1, fg1, fbe1, fw2, fb2, fg2, fbe2):
    BV = 200
    EB = BV * K
    grid = d2k.shape[0] // BV
    const = lambda shape: pl.BlockSpec(shape, lambda i: (0, 0))
    return pl.pallas_call(
        _edge_body,
        grid=(grid,),
        in_specs=[
            pl.BlockSpec((EB, DC), lambda i: (i, 0)),
            pl.BlockSpec((BV, K), lambda i: (i, 0)),
            pl.BlockSpec((BV, K), lambda i: (i, 0)),
            pl.BlockSpec((BV, H), lambda i: (i, 0)),
            const((2, GDF)),
            const((DC + 2 * GDF, H)), const((1, H)), const((1, H)), const((1, H)),
            const((H, 2 * H)), const((1, 2 * H)), const((1, 2 * H)), const((1, 2 * H)),
            const((2 * H, H)), const((1, H)), const((1, H)), const((1, H)),
            const((H, H)), const((1, H)), const((1, H)), const((1, H)),
        ],
        out_specs=pl.BlockSpec((BV, H), lambda i: (i, 0)),
        out_shape=jax.ShapeDtypeStruct((d2k.shape[0], H), jnp.float32),
    )(chem_g, d2k, angk, hg1, mu, sw1, sb1, sg1, sbe1, sw2, sb2, sg2,
      sbe2, fw1, fb1, fg1, fbe1, fw2, fb2, fg2, fbe2)


def kernel(chem_feats, geom_feats, verts, node_pos, cw1, cb1, cg1, cbe1, cw2,
           cb2, cg2, cbe2, gw1, gb1, gg1, gbe1, gw2, gb2, gg2, gbe2, sw1, sb1,
           sg1, sbe1, sw2, sb2, sg2, sbe2, fw1, fb1, fg1, fbe1, fw2, fb2, fg2,
           fbe2):
    row = lambda b: b.reshape(1, -1)
    h_chem, _ = _chem_prep(chem_feats, cw1, row(cb1), row(cg1), row(cbe1),
                           cw2, row(cb2), row(cg2), row(cbe2),
                           sw1[:DC], row(sb1))
    npT = jnp.pad(node_pos.T, ((0, 0), (0, NGP - NG)))
    pad = jnp.concatenate([jnp.zeros((1, NG), jnp.float32),
                           jnp.full((1, NGP - NG), 1e30, jnp.float32)], axis=1)
    _c = jnp.arange(NGP)
    permc = (2 * (_c % 80) + (_c // 80) // 16) * 16 + (_c // 80) % 16
    npTt = npT[:, permc]
    padt = pad[:, permc]
    hg1, d2, d2t, ang = _geom_prep(geom_feats, verts, npT, pad, npTt, padt,
                                   gw1, row(gb1), row(gg1), row(gbe1), gw2,
                                   row(gb2), row(gg2), row(gbe2))
    flat, d2e, ange = _sc_topk(d2.reshape(-1), d2t.reshape(-1),
                               ang.reshape(-1))
    tblc = lax.bitcast_convert_type(
        chem_feats.astype(jnp.bfloat16).reshape(NG, 64, 2), jnp.int32)
    mu = jnp.stack([jnp.linspace(0.0, 8.0, GDF),
                    jnp.linspace(-1.0, 1.0, GDF)]).astype(jnp.float32)
    d2k = d2e.reshape(NS, K)
    angk = ange.reshape(NS, K)
    NH = NS // 2
    halves = []
    for h2 in range(2):
        sl = slice(h2 * NH * K, (h2 + 1) * NH * K)
        sv = slice(h2 * NH, (h2 + 1) * NH)
        chem_i32 = _sc_gather(tblc, flat[sl])
        chem_g = lax.bitcast_convert_type(
            chem_i32, jnp.bfloat16).reshape(NH * K, DC)
        halves.append(_edge_final(
            chem_g, d2k[sv], angk[sv], hg1[sv], mu, sw1, row(sb1), row(sg1),
            row(sbe1), sw2, row(sb2), row(sg2), row(sbe2), fw1, row(fb1),
            row(fg1), row(fbe1), fw2, row(fb2), row(fg2), row(fbe2)))
    out = jnp.concatenate(halves, axis=0)
    return (out, h_chem)


# speedup vs baseline: 1.0206x; 1.0206x over previous
import functools
import math

import jax
import jax.numpy as jnp
from jax import lax
from jax.experimental import pallas as pl
from jax.experimental.pallas import tpu as pltpu
from jax.experimental.pallas import tpu_sc as plsc

NS, NG, DC, DG, H, GDF, K = 10000, 2500, 128, 16, 256, 16, 16
E = NS * K
BNS = 1.0 / math.sqrt(1.0 + 1e-5)

_SC_CORES, _SC_SUBCORES = 2, 16
_NW = _SC_CORES * _SC_SUBCORES


def _bn(x, g, b):
    return x * (BNS * g) + b


def _silu(x):
    return x * jax.nn.sigmoid(x)


def _chem_body(chem_ref, cw1_ref, cb1_ref, cg1_ref, cbe1_ref, cw2_ref, cb2_ref,
               cg2_ref, cbe2_ref, sw1c_ref, sb1_ref, hchem_ref, proj_ref):
    x = chem_ref[...]
    h = _bn(jnp.dot(x, cw1_ref[...], preferred_element_type=jnp.float32)
            + cb1_ref[...], cg1_ref[...], cbe1_ref[...])
    h = _silu(h)
    h = _bn(jnp.dot(h, cw2_ref[...], preferred_element_type=jnp.float32)
            + cb2_ref[...], cg2_ref[...], cbe2_ref[...])
    hchem_ref[...] = h
    proj_ref[...] = jnp.dot(x, sw1c_ref[...], preferred_element_type=jnp.float32)


def _chem_prep(chem_feats, cw1, cb1, cg1, cbe1, cw2, cb2, cg2, cbe2, sw1c, sb1):
    return pl.pallas_call(
        _chem_body,
        out_shape=(jax.ShapeDtypeStruct((NG, H), jnp.float32),
                   jax.ShapeDtypeStruct((NG, H), jnp.float32)),
    )(chem_feats, cw1, cb1, cg1, cbe1, cw2, cb2, cg2, cbe2, sw1c, sb1)


NGP = 2560


def _geom_body(geom_ref, verts_ref, npT_ref, pad_ref, npTt_ref, padt_ref,
               gw1_ref, gb1_ref, gg1_ref, gbe1_ref, gw2_ref, gb2_ref, gg2_ref,
               gbe2_ref, hg_ref, d2_ref, d2t_ref, ang_ref):
    x = geom_ref[...]
    h = _bn(jnp.dot(x, gw1_ref[...], preferred_element_type=jnp.float32)
            + gb1_ref[...], gg1_ref[...], gbe1_ref[...])
    h = _silu(h)
    h = _bn(jnp.dot(h, gw2_ref[...], preferred_element_type=jnp.float32)
            + gb2_ref[...], gg2_ref[...], gbe2_ref[...])
    hg_ref[...] = h
    v = verts_ref[...]
    npT = npT_ref[...]
    vsq = jnp.sum(v * v, axis=1, keepdims=True)
    nsq = jnp.sum(npT * npT, axis=0, keepdims=True) + pad_ref[...]
    d2_ref[...] = vsq + nsq - 2.0 * jnp.dot(v, npT,
                                            preferred_element_type=jnp.float32)
    npTt = npTt_ref[...]
    nsqt = jnp.sum(npTt * npTt, axis=0, keepdims=True) + padt_ref[...]
    d2t_ref[...] = vsq + nsqt - 2.0 * jnp.dot(v, npTt,
                                              preferred_element_type=jnp.float32)
    n = x[:, DG - 3:]
    nv = jnp.sum(n * v, axis=1, keepdims=True)
    ang_ref[...] = jnp.dot(n, npT,
                           preferred_element_type=jnp.float32) - nv


def _geom_prep(geom_feats, verts, npT, pad, npTt, padt, gw1, gb1, gg1, gbe1,
               gw2, gb2, gg2, gbe2):
    B = 400
    grid = NS // B
    return pl.pallas_call(
        _geom_body,
        grid=(grid,),
        in_specs=[
            pl.BlockSpec((B, DG), lambda i: (i, 0)),
            pl.BlockSpec((B, 3), lambda i: (i, 0)),
            pl.BlockSpec((3, NGP), lambda i: (0, 0)),
            pl.BlockSpec((1, NGP), lambda i: (0, 0)),
            pl.BlockSpec((3, NGP), lambda i: (0, 0)),
            pl.BlockSpec((1, NGP), lambda i: (0, 0)),
            pl.BlockSpec((DG, H), lambda i: (0, 0)),
            pl.BlockSpec((1, H), lambda i: (0, 0)),
            pl.BlockSpec((1, H), lambda i: (0, 0)),
            pl.BlockSpec((1, H), lambda i: (0, 0)),
            pl.BlockSpec((H, H), lambda i: (0, 0)),
            pl.BlockSpec((1, H), lambda i: (0, 0)),
            pl.BlockSpec((1, H), lambda i: (0, 0)),
            pl.BlockSpec((1, H), lambda i: (0, 0)),
        ],
        out_specs=[
            pl.BlockSpec((B, H), lambda i: (i, 0)),
            pl.BlockSpec((B, NGP), lambda i: (i, 0)),
            pl.BlockSpec((B, NGP), lambda i: (i, 0)),
            pl.BlockSpec((B, NGP), lambda i: (i, 0)),
        ],
        out_shape=(jax.ShapeDtypeStruct((NS, H), jnp.float32),
                   jax.ShapeDtypeStruct((NS, NGP), jnp.float32),
                   jax.ShapeDtypeStruct((NS, NGP), jnp.float32),
                   jax.ShapeDtypeStruct((NS, NGP), jnp.float32)),
    )(geom_feats, verts, npT, pad, npTt, padt, gw1, gb1, gg1, gbe1, gw2, gb2,
      gg2, gbe2)


_TG = 4
_NGRP = NS // _TG
_CPV = NGP // 16


def _lexmin(v1, c1, v2, c2):
    take = (v2 < v1) | ((v2 == v1) & (c2 < c1))
    return jnp.where(take, v2, v1), jnp.where(take, c2, c1)


_CL = _CPV // 2


def _sc_topk_body(d2_ref, d2t_ref, ang_ref, idx_ref, d2e_ref, ange_ref,
                  buf0, buft0, bufa0, buf1, buft1, bufa1,
                  stage, staged, stagea,
                  sem0, semt0, sema0, sem1, semt1, sema1):
    wid = lax.axis_index("s") * _SC_CORES + lax.axis_index("c")
    lane = lax.iota(jnp.int32, 16)
    inf16 = jnp.full((16,), jnp.inf, jnp.float32)
    big16 = jnp.full((16,), NGP, jnp.int32)
    sets = ((buf0, buft0, bufa0, sem0, semt0, sema0),
            (buf1, buft1, bufa1, sem1, semt1, sema1))

    def start(gi, p):
        buf, buft, bufa, sem, semt, sema = sets[p]
        pltpu.async_copy(d2_ref.at[pl.ds(gi * (_TG * NGP), _TG * NGP)], buf,
                         sem)
        pltpu.async_copy(d2t_ref.at[pl.ds(gi * (_TG * NGP), _TG * NGP)], buft,
                         semt)
        pltpu.async_copy(ang_ref.at[pl.ds(gi * (_TG * NGP), _TG * NGP)], bufa,
                         sema)

    def do_group(gi, p):
        buf, buft, bufa, sem, semt, sema = sets[p]
        pltpu.make_async_copy(
            d2_ref.at[pl.ds(gi * (_TG * NGP), _TG * NGP)], buf, sem).wait()
        pltpu.make_async_copy(
            d2t_ref.at[pl.ds(gi * (_TG * NGP), _TG * NGP)], buft, semt).wait()
        pltpu.make_async_copy(
            ang_ref.at[pl.ds(gi * (_TG * NGP), _TG * NGP)], bufa, sema).wait()
        def do_row(rr, rcarry):
            roff = rr * NGP

            def p1(t, carry):
                a0, s0, a1, s1 = carry
                j0 = 4 * t
                for u in (0, 2):
                    v0 = buf[pl.ds(roff + (j0 + u) * 16, 16)]
                    c0 = v0 < a0
                    a0 = jnp.where(c0, v0, a0)
                    s0 = jnp.where(c0, j0 + u, s0)
                    v1 = buf[pl.ds(roff + (j0 + u + 1) * 16, 16)]
                    c1 = v1 < a1
                    a1 = jnp.where(c1, v1, a1)
                    s1 = jnp.where(c1, j0 + u + 1, s1)
                return a0, s0, a1, s1

            a0, s0, a1, s1 = lax.fori_loop(
                0, _CPV // 4, p1, (inf16, big16, inf16, big16))

            def sel(k, carry):
                a0, s0, a1, s1, outv, outd, outa = carry
                val, col = _lexmin(a0, s0 * 16 + lane, a1, s1 * 16 + lane)
                for s in (1, 2, 4, 8):
                    pp = lane ^ s
                    v2 = val.at[pp].get(mode="promise_in_bounds")
                    c2 = col.at[pp].get(mode="promise_in_bounds")
                    val, col = _lexmin(val, col, v2, c2)
                outv = jnp.where(lane == k, col, outv)
                outd = jnp.where(lane == k, val, outd)
                av = bufa[pl.ds(roff + (col[0] >> 4) * 16, 16)]
                al = av.at[col & 15].get(mode="promise_in_bounds")
                outa = jnp.where(lane == k, al, outa)
                ls = col[0] & 15
                jcol = col >> 4
                ps = col[0] >> 4 & 1
                coff = roff + (ps * 16 + ls) * _CL

                def rf(t, carry2):
                    a, aj = carry2
                    w = buft[pl.ds(coff + t * 16, 16)]
                    jv = (t * 16 + lane) * 2 + ps
                    elig = (w > val) | ((w == val) & (jv > jcol))
                    w = jnp.where(elig, w, jnp.inf)
                    return _lexmin(a, aj, w, jv)

                a, aj = lax.fori_loop(0, _CL // 16, rf, (inf16, big16))
                for s in (1, 2, 4, 8):
                    pp = lane ^ s
                    v2 = a.at[pp].get(mode="promise_in_bounds")
                    c2 = aj.at[pp].get(mode="promise_in_bounds")
                    a, aj = _lexmin(a, aj, v2, c2)
                upd = jnp.where(lane == ls, ps + 1, 0)
                a0 = jnp.where(upd == 1, a, a0)
                s0 = jnp.where(upd == 1, aj, s0)
                a1 = jnp.where(upd == 2, a, a1)
                s1 = jnp.where(upd == 2, aj, s1)
                return a0, s0, a1, s1, outv, outd, outa

            carry = (a0, s0, a1, s1, jnp.zeros((16,), jnp.int32),
                     jnp.zeros((16,), jnp.float32), jnp.zeros((16,), jnp.float32))
            res = lax.fori_loop(0, K, sel, carry)
            stage[pl.ds(rr * K, K)] = res[4]
            staged[pl.ds(rr * K, K)] = res[5]
            stagea[pl.ds(rr * K, K)] = res[6]
            return rcarry

        lax.fori_loop(0, _TG, do_row, 0)
        pltpu.sync_copy(stage, idx_ref.at[pl.ds(gi * (_TG * K), _TG * K)])
        pltpu.sync_copy(staged, d2e_ref.at[pl.ds(gi * (_TG * K), _TG * K)])
        pltpu.sync_copy(stagea, ange_ref.at[pl.ds(gi * (_TG * K), _TG * K)])

    nloop = (_NGRP + _NW - 1) // _NW

    @pl.when(wid < _NGRP)
    def _():
        start(wid, 0)

    def loop_body(t, carry):
        for p in (0, 1):
            j = 2 * t + p
            gi = wid + j * _NW
            gin = wid + (j + 1) * _NW

            @pl.when(gin < _NGRP)
            def _():
                start(gin, 1 - p)

            @pl.when(gi < _NGRP)
            def _():
                do_group(gi, p)

        return carry

    lax.fori_loop(0, (nloop + 1) // 2, loop_body, 0)


def _sc_topk(d2_flat, d2t_flat, ang_flat):
    return pl.kernel(
        _sc_topk_body,
        out_type=(jax.ShapeDtypeStruct((E,), jnp.int32),
                  jax.ShapeDtypeStruct((E,), jnp.float32),
                  jax.ShapeDtypeStruct((E,), jnp.float32)),
        mesh=plsc.VectorSubcoreMesh(core_axis_name="c", subcore_axis_name="s",
                                    num_cores=_SC_CORES,
                                    num_subcores=_SC_SUBCORES),
        scratch_types=[pltpu.VMEM((_TG * NGP,), jnp.float32),
                       pltpu.VMEM((_TG * NGP,), jnp.float32),
                       pltpu.VMEM((_TG * NGP,), jnp.float32),
                       pltpu.VMEM((_TG * NGP,), jnp.float32),
                       pltpu.VMEM((_TG * NGP,), jnp.float32),
                       pltpu.VMEM((_TG * NGP,), jnp.float32),
                       pltpu.VMEM((_TG * K,), jnp.int32),
                       pltpu.VMEM((_TG * K,), jnp.float32),
                       pltpu.VMEM((_TG * K,), jnp.float32),
                       pltpu.SemaphoreType.DMA,
                       pltpu.SemaphoreType.DMA,
                       pltpu.SemaphoreType.DMA,
                       pltpu.SemaphoreType.DMA,
                       pltpu.SemaphoreType.DMA,
                       pltpu.SemaphoreType.DMA],
        compiler_params=pltpu.CompilerParams(use_tc_tiling_on_sc=False),
    )(d2_flat, d2t_flat, ang_flat)


_GC = 128


def _sc_gather_body(tblc_ref, idx_ref, chem_ref, idx_v, chem_v, sem1):
    wid = lax.axis_index("s") * _SC_CORES + lax.axis_index("c")
    nchunk = E // _GC

    def do_chunk(ci):
        base = ci * _GC
        pltpu.sync_copy(idx_ref.at[pl.ds(base, _GC)], idx_v)
        pltpu.async_copy(tblc_ref.at[idx_v], chem_v, sem1).wait()
        pltpu.sync_copy(chem_v, chem_ref.at[pl.ds(base, _GC)])

    def loop_body(j, carry):
        do_chunk(wid + j * _NW)
        return carry

    lax.fori_loop(0, nchunk // _NW, loop_body, 0)
    rem = nchunk - (nchunk // _NW) * _NW

    @pl.when(wid < rem)
    def _():
        do_chunk((nchunk // _NW) * _NW + wid)


def _sc_gather(tblc, idx_flat):
    return pl.kernel(
        _sc_gather_body,
        out_type=jax.ShapeDtypeStruct((E, 64), jnp.int32),
        mesh=plsc.VectorSubcoreMesh(core_axis_name="c", subcore_axis_name="s",
                                    num_cores=_SC_CORES,
                                    num_subcores=_SC_SUBCORES),
        scratch_types=[pltpu.VMEM((_GC,), jnp.int32),
                       pltpu.VMEM((_GC, 64), jnp.int32),
                       pltpu.SemaphoreType.DMA],
        compiler_params=pltpu.CompilerParams(use_tc_tiling_on_sc=False),
    )(tblc, idx_flat)


def _edge_body(chem_g_ref, d2k_ref, angk_ref, hg1_ref, mu_ref,
               sw1_ref, sb1_ref, sg1_ref, sbe1_ref, sw2_ref, sb2_ref,
               sg2_ref, sbe2_ref, fw1_ref, fb1_ref, fg1_ref, fbe1_ref,
               fw2_ref, fb2_ref, fg2_ref, fbe2_ref, out_ref):
    BV = d2k_ref.shape[0]
    EB = BV * K

    dist = jnp.sqrt(jnp.maximum(d2k_ref[...], 1e-30))
    ang = angk_ref[...] / dist

    mu_d = mu_ref[0:1, :].reshape(1, 1, GDF)
    mu_a = mu_ref[1:2, :].reshape(1, 1, GDF)
    enc_d = jnp.exp(-(((dist[:, :, None] - mu_d) / 0.5) ** 2)).reshape(EB, GDF)
    enc_a = jnp.exp(-(((ang[:, :, None] - mu_a) / 0.125) ** 2)).reshape(EB, GDF)

    enc = jnp.concatenate([enc_d, enc_a], axis=1)
    x1 = (jnp.dot(chem_g_ref[...], sw1_ref[:DC, :].astype(jnp.bfloat16),
                  preferred_element_type=jnp.float32)
          + jnp.dot(enc, sw1_ref[DC:, :], preferred_element_type=jnp.float32))
    h = _bn(x1 + sb1_ref[...], sg1_ref[...], sbe1_ref[...])
    h = _silu(h)
    h = _bn(jnp.dot(h.astype(jnp.bfloat16), sw2_ref[...].astype(jnp.bfloat16),
                    preferred_element_type=jnp.float32)
            + sb2_ref[...], sg2_ref[...], sbe2_ref[...])
    filt, core = h[:, :H], h[:, H:]
    he = jax.nn.sigmoid(filt) * jax.nn.softplus(core)
    h_cg = jnp.sum(he.reshape(BV, K, H), axis=1)

    y = jnp.concatenate([h_cg, hg1_ref[...]], axis=1)
    y = _bn(jnp.dot(y, fw1_ref[...], preferred_element_type=jnp.float32)
            + fb1_ref[...], fg1_ref[...], fbe1_ref[...])
    y = _silu(y)
    y = _bn(jnp.dot(y, fw2_ref[...], preferred_element_type=jnp.float32)
            + fb2_ref[...], fg2_ref[...], fbe2_ref[...])
    out_ref[...] = y


def _edge_final(chem_g, d2k, angk, hg1, mu, sw1, sb1, sg1, sbe1, sw2,
                sb2, sg2, sbe2, fw1, fb1, fg1, fbe1, fw2, fb2, fg2, fbe2):
    BV = 400
    EB = BV * K
    grid = NS // BV
    const = lambda shape: pl.BlockSpec(shape, lambda i: (0, 0))
    return pl.pallas_call(
        _edge_body,
        grid=(grid,),
        in_specs=[
            pl.BlockSpec((EB, DC), lambda i: (i, 0)),
            pl.BlockSpec((BV, K), lambda i: (i, 0)),
            pl.BlockSpec((BV, K), lambda i: (i, 0)),
            pl.BlockSpec((BV, H), lambda i: (i, 0)),
            const((2, GDF)),
            const((DC + 2 * GDF, H)), const((1, H)), const((1, H)), const((1, H)),
            const((H, 2 * H)), const((1, 2 * H)), const((1, 2 * H)), const((1, 2 * H)),
            const((2 * H, H)), const((1, H)), const((1, H)), const((1, H)),
            const((H, H)), const((1, H)), const((1, H)), const((1, H)),
        ],
        out_specs=pl.BlockSpec((BV, H), lambda i: (i, 0)),
        out_shape=jax.ShapeDtypeStruct((NS, H), jnp.float32),
    )(chem_g, d2k, angk, hg1, mu, sw1, sb1, sg1, sbe1, sw2, sb2, sg2,
      sbe2, fw1, fb1, fg1, fbe1, fw2, fb2, fg2, fbe2)


def kernel(chem_feats, geom_feats, verts, node_pos, cw1, cb1, cg1, cbe1, cw2,
           cb2, cg2, cbe2, gw1, gb1, gg1, gbe1, gw2, gb2, gg2, gbe2, sw1, sb1,
           sg1, sbe1, sw2, sb2, sg2, sbe2, fw1, fb1, fg1, fbe1, fw2, fb2, fg2,
           fbe2):
    row = lambda b: b.reshape(1, -1)
    h_chem, _ = _chem_prep(chem_feats, cw1, row(cb1), row(cg1), row(cbe1),
                           cw2, row(cb2), row(cg2), row(cbe2),
                           sw1[:DC], row(sb1))
    npT = jnp.pad(node_pos.T, ((0, 0), (0, NGP - NG)))
    pad = jnp.concatenate([jnp.zeros((1, NG), jnp.float32),
                           jnp.full((1, NGP - NG), 1e30, jnp.float32)], axis=1)
    _c = jnp.arange(NGP)
    permc = (2 * (_c % 80) + (_c // 80) // 16) * 16 + (_c // 80) % 16
    npTt = npT[:, permc]
    padt = pad[:, permc]
    hg1, d2, d2t, ang = _geom_prep(geom_feats, verts, npT, pad, npTt, padt,
                                   gw1, row(gb1), row(gg1), row(gbe1), gw2,
                                   row(gb2), row(gg2), row(gbe2))
    flat, d2e, ange = _sc_topk(d2.reshape(-1), d2t.reshape(-1),
                               ang.reshape(-1))
    tblc = lax.bitcast_convert_type(
        chem_feats.astype(jnp.bfloat16).reshape(NG, 64, 2), jnp.int32)
    chem_i32 = _sc_gather(tblc, flat)
    chem_g = lax.bitcast_convert_type(chem_i32, jnp.bfloat16).reshape(E, DC)
    mu = jnp.stack([jnp.linspace(0.0, 8.0, GDF),
                    jnp.linspace(-1.0, 1.0, GDF)]).astype(jnp.float32)
    out = _edge_final(chem_g, d2e.reshape(NS, K), ange.reshape(NS, K), hg1,
                      mu, sw1, row(sb1), row(sg1), row(sbe1), sw2, row(sb2),
                      row(sg2), row(sbe2), fw1, row(fb1), row(fg1), row(fbe1),
                      fw2, row(fb2), row(fg2), row(fbe2))
    return (out, h_chem)
